# baseline (speedup 1.0000x reference)
import functools

import jax
import jax.numpy as jnp
from jax import lax
from jax.experimental import pallas as pl
from jax.experimental.pallas import tpu as pltpu
from jax.experimental.pallas import tpu_sc as plsc

EPS = 1e-15
MIN_NORM = 1e-15
R = 2.0
T = 1.0

NC, NS, L = 2, 16, 16
NW = NC * NS
D = 128
CHUNK = 80
GROUPS = CHUNK // L
CPB = 10


def _sc_dots(n_edges, n_nodes):
    e_per_w = n_edges // NW
    rows_per_w = e_per_w // CHUNK
    n_blocks = rows_per_w // CPB
    assert e_per_w % CHUNK == 0 and rows_per_w % CPB == 0
    BLK = CPB * CHUNK

    mesh = plsc.VectorSubcoreMesh(core_axis_name="c", subcore_axis_name="s")

    @functools.partial(
        pl.kernel,
        out_type=[jax.ShapeDtypeStruct((n_edges,), jnp.float32)] * 3,
        mesh=mesh,
        compiler_params=pltpu.CompilerParams(needs_layout_passes=False),
        scratch_types=[
            pltpu.VMEM((BLK,), jnp.int32),
            pltpu.VMEM((BLK,), jnp.int32),
            pltpu.VMEM((CHUNK, 2 * D), jnp.float32),
            pltpu.VMEM((CHUNK, 2 * D), jnp.float32),
            pltpu.VMEM((CHUNK, 2 * D), jnp.float32),
            pltpu.VMEM((CHUNK, 2 * D), jnp.float32),
            pltpu.VMEM((BLK,), jnp.float32),
            pltpu.VMEM((BLK,), jnp.float32),
            pltpu.VMEM((BLK,), jnp.float32),
            pltpu.SemaphoreType.DMA,
            pltpu.SemaphoreType.DMA,
        ],
    )
    def body(z_hbm, ei_hbm, ej_hbm, x2_hbm, y2_hbm, xy_hbm,
             idx_i, idx_j, ri0, ri1, rj0, rj1, sx2, sy2, sxy, sem0, sem1):
        wid = lax.axis_index("s") * NC + lax.axis_index("c")
        wbase = wid * e_per_w

        def compute_chunk(ri, rj, row):
            def group_body(g, gcarry):
                lanes = lax.iota(jnp.int32, L)
                e_idx = g * L + lanes
                st = pl.ds(row * CHUNK + g * L, L)
                rot = [(lanes + dd) & (L - 1) for dd in range(L)]
                acc = [jnp.zeros((L,), jnp.float32) for _ in range(8)]
                for d in range(D):
                    k = d & 7
                    d_idx = (d & ~(L - 1)) + rot[d & (L - 1)]
                    xi = plsc.load_gather(ri, [e_idx, d_idx])
                    xj = plsc.load_gather(rj, [e_idx, d_idx])
                    acc[k] = acc[k] + xi * xj
                sxy[st] = ((acc[0] + acc[1]) + (acc[2] + acc[3])) + \
                          ((acc[4] + acc[5]) + (acc[6] + acc[7]))
                ncol = lanes * 0 + D
                sx2[st] = plsc.load_gather(ri, [e_idx, ncol])
                sy2[st] = plsc.load_gather(rj, [e_idx, ncol])
                return gcarry

            lax.fori_loop(0, GROUPS, group_body, 0)

        def chunk_copies(ri, rj, row):
            sl_i = idx_i.at[pl.ds(row * CHUNK, CHUNK)]
            sl_j = idx_j.at[pl.ds(row * CHUNK, CHUNK)]
            return (
                (z_hbm.at[sl_i], ri),
                (z_hbm.at[sl_j], rj),
            )

        def gather_chunk(ri, rj, row, sem):
            for src, dst in chunk_copies(ri, rj, row):
                pltpu.async_copy(src, dst, sem)

        def wait_chunk(ri, rj, row, sem):
            for src, dst in chunk_copies(ri, rj, row):
                pltpu.make_async_copy(src, dst, sem).wait()

        def blk_body(b, carry):
            base = wbase + b * BLK
            pltpu.sync_copy(ei_hbm.at[pl.ds(base, BLK)], idx_i)
            pltpu.sync_copy(ej_hbm.at[pl.ds(base, BLK)], idx_j)
            gather_chunk(ri0, rj0, 0, sem0)

            def pair_body(t, pcarry):
                r0 = 2 * t
                r1 = r0 + 1
                gather_chunk(ri1, rj1, r1, sem1)
                wait_chunk(ri0, rj0, r0, sem0)
                compute_chunk(ri0, rj0, r0)

                @pl.when(t < CPB // 2 - 1)
                def _():
                    gather_chunk(ri0, rj0, r0 + 2, sem0)

                wait_chunk(ri1, rj1, r1, sem1)
                compute_chunk(ri1, rj1, r1)
                return pcarry

            lax.fori_loop(0, CPB // 2, pair_body, 0)
            pltpu.sync_copy(sx2, x2_hbm.at[pl.ds(base, BLK)])
            pltpu.sync_copy(sy2, y2_hbm.at[pl.ds(base, BLK)])
            pltpu.sync_copy(sxy, xy_hbm.at[pl.ds(base, BLK)])
            return carry

        lax.fori_loop(0, n_blocks, blk_body, 0)

    return body


def _tc_norms_kernel(z_ref, out_ref):
    z = z_ref[...]
    out_ref[...] = jnp.sum(z * z, axis=1, keepdims=True)


def _tc_loss_kernel(n_pos_rows, n_pos, n_neg, alpha,
                    x2_ref, y2_ref, xy_ref, out_ref):
    x2 = x2_ref[...]
    y2 = y2_ref[...]
    dot = xy_ref[...]
    a = (1.0 + y2) - 2.0 * dot
    b = 1.0 - x2
    num2 = jnp.maximum(a * a * x2 - 2.0 * a * b * dot + b * b * y2, 0.0)
    den = jnp.maximum((1.0 + x2 * y2) - 2.0 * dot, MIN_NORM)
    norm = jnp.sqrt(num2) / den
    u = jnp.clip(norm, -1.0 + 1e-7, 1.0 - 1e-7)
    dist = jnp.log1p(u) - jnp.log1p(-u)
    d2 = dist * dist
    prob = 1.0 / (jnp.exp((d2 - R) / T) + 1.0)
    pos_terms = -jnp.log(prob + EPS)
    neg_terms = -jnp.log((1.0 - prob) + EPS)
    pos_sum = jnp.sum(pos_terms[:n_pos_rows, :])
    neg_sum = jnp.sum(neg_terms[n_pos_rows:, :])
    out_ref[0, 0] = pos_sum * (alpha / n_pos) + neg_sum / n_neg


def kernel(z, pos_edge_index, neg_edge_index):
    n_pos = pos_edge_index.shape[1]
    n_neg = neg_edge_index.shape[1]
    alpha = n_neg / n_pos
    pe = pos_edge_index.astype(jnp.int32)
    ne = neg_edge_index.astype(jnp.int32)
    n_edges = n_pos + n_neg
    ei = jnp.concatenate([pe[0], ne[0]])
    ej = jnp.concatenate([pe[1], ne[1]])

    zf = z.astype(jnp.float32)
    n_nodes = zf.shape[0]

    norms = pl.pallas_call(
        _tc_norms_kernel,
        out_shape=jax.ShapeDtypeStruct((n_nodes, 1), jnp.float32),
    )(zf)

    zaug = jnp.concatenate(
        [zf, norms, jnp.zeros((n_nodes, D - 1), jnp.float32)], axis=1)

    x2, y2, xy = _sc_dots(n_edges, n_nodes)(zaug, ei, ej)

    tc_rows = n_edges // D
    n_pos_rows = n_pos // D
    out = pl.pallas_call(
        functools.partial(_tc_loss_kernel, n_pos_rows, n_pos, n_neg, alpha),
        out_shape=jax.ShapeDtypeStruct((1, 1), jnp.float32),
        out_specs=pl.BlockSpec(memory_space=pltpu.SMEM),
    )(x2.reshape(tc_rows, D), y2.reshape(tc_rows, D), xy.reshape(tc_rows, D))
    return out[0, 0]


# speedup vs baseline: 1.9429x; 1.9429x over previous
import functools

import jax
import jax.numpy as jnp
from jax import lax
from jax.experimental import pallas as pl
from jax.experimental.pallas import tpu as pltpu
from jax.experimental.pallas import tpu_sc as plsc

EPS = 1e-15
MIN_NORM = 1e-15
R = 2.0
T = 1.0

NC, NS, L = 2, 16, 16
NW = NC * NS
D = 128
CHUNK = 80
GROUPS = CHUNK // L
CPB = 50


def _sc_dots(n_edges):
    e_per_w = n_edges // NW
    rows_per_w = e_per_w // CHUNK
    n_blocks = rows_per_w // CPB
    assert e_per_w % CHUNK == 0 and rows_per_w % CPB == 0
    BLK = CPB * CHUNK

    mesh = plsc.VectorSubcoreMesh(core_axis_name="c", subcore_axis_name="s")

    @functools.partial(
        pl.kernel,
        out_type=[jax.ShapeDtypeStruct((n_edges,), jnp.float32)] * 3,
        mesh=mesh,
        compiler_params=pltpu.CompilerParams(needs_layout_passes=False),
        scratch_types=[
            pltpu.VMEM((BLK,), jnp.int32),
            pltpu.VMEM((BLK,), jnp.int32),
            pltpu.VMEM((CHUNK, D), jnp.float32),
            pltpu.VMEM((CHUNK, D), jnp.float32),
            pltpu.VMEM((CHUNK, D), jnp.float32),
            pltpu.VMEM((CHUNK, D), jnp.float32),
            pltpu.VMEM((BLK,), jnp.float32),
            pltpu.VMEM((BLK,), jnp.float32),
            pltpu.VMEM((BLK,), jnp.float32),
            pltpu.SemaphoreType.DMA,
            pltpu.SemaphoreType.DMA,
        ],
    )
    def body(z_hbm, ei_hbm, ej_hbm, x2_hbm, y2_hbm, xy_hbm,
             idx_i, idx_j, ri0, ri1, rj0, rj1, sx2, sy2, sxy, sem0, sem1):
        wid = lax.axis_index("s") * NC + lax.axis_index("c")
        wbase = wid * e_per_w

        def compute_chunk(ri, rj, row):
            def group_body(g, gcarry):
                lanes = lax.iota(jnp.int32, L)
                e_idx = g * L + lanes
                st = pl.ds(row * CHUNK + g * L, L)
                rot = [(lanes + dd) & (L - 1) for dd in range(L)]
                axy = [jnp.zeros((L,), jnp.float32) for _ in range(4)]
                axx = [jnp.zeros((L,), jnp.float32) for _ in range(4)]
                ayy = [jnp.zeros((L,), jnp.float32) for _ in range(4)]
                for d in range(D):
                    k = d & 3
                    d_idx = (d & ~(L - 1)) + rot[d & (L - 1)]
                    xi = plsc.load_gather(ri, [e_idx, d_idx])
                    xj = plsc.load_gather(rj, [e_idx, d_idx])
                    axy[k] = axy[k] + xi * xj
                    axx[k] = axx[k] + xi * xi
                    ayy[k] = ayy[k] + xj * xj
                sxy[st] = (axy[0] + axy[1]) + (axy[2] + axy[3])
                sx2[st] = (axx[0] + axx[1]) + (axx[2] + axx[3])
                sy2[st] = (ayy[0] + ayy[1]) + (ayy[2] + ayy[3])
                return gcarry

            lax.fori_loop(0, GROUPS, group_body, 0)

        def chunk_copies(ri, rj, row):
            sl_i = idx_i.at[pl.ds(row * CHUNK, CHUNK)]
            sl_j = idx_j.at[pl.ds(row * CHUNK, CHUNK)]
            return (
                (z_hbm.at[sl_i], ri),
                (z_hbm.at[sl_j], rj),
            )

        def gather_chunk(ri, rj, row, sem):
            for src, dst in chunk_copies(ri, rj, row):
                pltpu.async_copy(src, dst, sem)

        def wait_chunk(ri, rj, row, sem):
            for src, dst in chunk_copies(ri, rj, row):
                pltpu.make_async_copy(src, dst, sem).wait()

        def blk_body(b, carry):
            base = wbase + b * BLK
            pltpu.sync_copy(ei_hbm.at[pl.ds(base, BLK)], idx_i)
            pltpu.sync_copy(ej_hbm.at[pl.ds(base, BLK)], idx_j)
            gather_chunk(ri0, rj0, 0, sem0)

            def pair_body(t, pcarry):
                r0 = 2 * t
                r1 = r0 + 1
                gather_chunk(ri1, rj1, r1, sem1)
                wait_chunk(ri0, rj0, r0, sem0)
                compute_chunk(ri0, rj0, r0)

                @pl.when(t < CPB // 2 - 1)
                def _():
                    gather_chunk(ri0, rj0, r0 + 2, sem0)

                wait_chunk(ri1, rj1, r1, sem1)
                compute_chunk(ri1, rj1, r1)
                return pcarry

            lax.fori_loop(0, CPB // 2, pair_body, 0)
            pltpu.sync_copy(sx2, x2_hbm.at[pl.ds(base, BLK)])
            pltpu.sync_copy(sy2, y2_hbm.at[pl.ds(base, BLK)])
            pltpu.sync_copy(sxy, xy_hbm.at[pl.ds(base, BLK)])
            return carry

        lax.fori_loop(0, n_blocks, blk_body, 0)

    return body


def _tc_loss_kernel(n_pos_rows, n_pos, n_neg, alpha,
                    x2_ref, y2_ref, xy_ref, out_ref):
    x2 = x2_ref[...]
    y2 = y2_ref[...]
    dot = xy_ref[...]
    a = (1.0 + y2) - 2.0 * dot
    b = 1.0 - x2
    num2 = jnp.maximum(a * a * x2 - 2.0 * a * b * dot + b * b * y2, 0.0)
    den = jnp.maximum((1.0 + x2 * y2) - 2.0 * dot, MIN_NORM)
    norm = jnp.sqrt(num2) / den
    u = jnp.clip(norm, -1.0 + 1e-7, 1.0 - 1e-7)
    dist = jnp.log1p(u) - jnp.log1p(-u)
    d2 = dist * dist
    prob = 1.0 / (jnp.exp((d2 - R) / T) + 1.0)
    pos_terms = -jnp.log(prob + EPS)
    neg_terms = -jnp.log((1.0 - prob) + EPS)
    pos_sum = jnp.sum(pos_terms[:n_pos_rows, :])
    neg_sum = jnp.sum(neg_terms[n_pos_rows:, :])
    out_ref[0, 0] = pos_sum * (alpha / n_pos) + neg_sum / n_neg


def kernel(z, pos_edge_index, neg_edge_index):
    n_pos = pos_edge_index.shape[1]
    n_neg = neg_edge_index.shape[1]
    alpha = n_neg / n_pos
    pe = pos_edge_index.astype(jnp.int32)
    ne = neg_edge_index.astype(jnp.int32)
    n_edges = n_pos + n_neg
    ei = jnp.concatenate([pe[0], ne[0]])
    ej = jnp.concatenate([pe[1], ne[1]])

    zf = z.astype(jnp.float32)

    x2, y2, xy = _sc_dots(n_edges)(zf, ei, ej)

    tc_rows = n_edges // D
    n_pos_rows = n_pos // D
    out = pl.pallas_call(
        functools.partial(_tc_loss_kernel, n_pos_rows, n_pos, n_neg, alpha),
        out_shape=jax.ShapeDtypeStruct((1, 1), jnp.float32),
        out_specs=pl.BlockSpec(memory_space=pltpu.SMEM),
    )(x2.reshape(tc_rows, D), y2.reshape(tc_rows, D), xy.reshape(tc_rows, D))
    return out[0, 0]
